# baseline (speedup 1.0000x reference)
import functools

import jax
import jax.numpy as jnp
from jax import lax
from jax.experimental import pallas as pl
from jax.experimental.pallas import tpu as pltpu
from jax.experimental.pallas import tpu_sc as plsc

L = 200
B = 1024
V = 1000000
D = 64
N = L * B

_info = plsc.get_sparse_core_info()
NC = _info.num_cores
NS = _info.num_subcores
NW = NC * NS
PER_W = N // NW
CHUNK = 640
NCHUNK = PER_W // CHUNK


def _gather_body(embs_hbm, i0_hbm, i1_hbm, i2_hbm,
                 o0_hbm, o1_hbm, o2_hbm,
                 idx_v, rows_v, sem):
    wid = lax.axis_index("s") * NC + lax.axis_index("c")
    base = wid * PER_W
    for idx_hbm, out_hbm in ((i0_hbm, o0_hbm), (i1_hbm, o1_hbm),
                             (i2_hbm, o2_hbm)):
        def body(k, _, idx_hbm=idx_hbm, out_hbm=out_hbm):
            off = base + k * CHUNK
            pltpu.sync_copy(idx_hbm.at[pl.ds(off, CHUNK)], idx_v)
            pltpu.async_copy(embs_hbm.at[idx_v], rows_v, sem).wait()
            pltpu.sync_copy(rows_v, out_hbm.at[pl.ds(off, CHUNK)])
            return ()
        lax.fori_loop(0, NCHUNK, body, ())


_mesh = plsc.VectorSubcoreMesh(core_axis_name="c", subcore_axis_name="s")

_gather3 = functools.partial(
    pl.kernel,
    out_type=(
        jax.ShapeDtypeStruct((N, D), jnp.float32),
        jax.ShapeDtypeStruct((N, D), jnp.float32),
        jax.ShapeDtypeStruct((N, D), jnp.float32),
    ),
    mesh=_mesh,
    compiler_params=pltpu.CompilerParams(use_tc_tiling_on_sc=False),
    scratch_types=[
        pltpu.VMEM((CHUNK,), jnp.int32),
        pltpu.VMEM((CHUNK, D), jnp.float32),
        pltpu.SemaphoreType.DMA,
    ],
)(_gather_body)


def kernel(embs, idx0, idx1, idx2, p0, p1, p2, len0, len1, len2):
    o0, o1, o2 = _gather3(embs, idx0.reshape(N), idx1.reshape(N),
                          idx2.reshape(N))
    return (o1.reshape(L, B, D), p1, len1,
            o2.reshape(L, B, D), p2, len2,
            o0.reshape(L, B, D), p0, len0)


# speedup vs baseline: 1.0291x; 1.0291x over previous
import functools

import jax
import jax.numpy as jnp
from jax import lax
from jax.experimental import pallas as pl
from jax.experimental.pallas import tpu as pltpu
from jax.experimental.pallas import tpu_sc as plsc

L = 200
B = 1024
V = 1000000
D = 64
N = L * B

_info = plsc.get_sparse_core_info()
NC = _info.num_cores
NS = _info.num_subcores
NW = NC * NS
PER_W = N // NW
CHUNK = 400
NCHUNK = PER_W // CHUNK
NBUF = 3
TOTAL = 3 * NCHUNK


def _gather_body(embs_hbm, i0_hbm, i1_hbm, i2_hbm,
                 o0_hbm, o1_hbm, o2_hbm,
                 idx_v, rows, gsems, ssems):
    wid = lax.axis_index("s") * NC + lax.axis_index("c")
    base = wid * PER_W
    for a, idx_hbm in enumerate((i0_hbm, i1_hbm, i2_hbm)):
        pltpu.sync_copy(idx_hbm.at[pl.ds(base, PER_W)],
                        idx_v.at[pl.ds(a * PER_W, PER_W)])

    outs = (o0_hbm, o1_hbm, o2_hbm)
    g_desc, s_desc = {}, {}

    def start_gather(t):
        a, k = divmod(t, NCHUNK)
        b = t % NBUF
        src = embs_hbm.at[idx_v.at[pl.ds(a * PER_W + k * CHUNK, CHUNK)]]
        g_desc[t] = pltpu.async_copy(src, rows.at[b], gsems.at[b])

    def start_store(t):
        a, k = divmod(t, NCHUNK)
        b = t % NBUF
        dst = outs[a].at[pl.ds(base + k * CHUNK, CHUNK)]
        s_desc[t] = pltpu.async_copy(rows.at[b], dst, ssems.at[b])

    for t in range(min(NBUF - 1, TOTAL)):
        start_gather(t)
    for t in range(TOTAL):
        g_desc[t].wait()
        start_store(t)
        nxt = t + NBUF - 1
        if nxt < TOTAL:
            if nxt >= NBUF:
                s_desc[nxt - NBUF].wait()
            start_gather(nxt)
    for t in range(max(0, TOTAL - NBUF), TOTAL):
        s_desc[t].wait()


_mesh = plsc.VectorSubcoreMesh(core_axis_name="c", subcore_axis_name="s")

_gather3 = functools.partial(
    pl.kernel,
    out_type=(
        jax.ShapeDtypeStruct((N, D), jnp.float32),
        jax.ShapeDtypeStruct((N, D), jnp.float32),
        jax.ShapeDtypeStruct((N, D), jnp.float32),
    ),
    mesh=_mesh,
    compiler_params=pltpu.CompilerParams(use_tc_tiling_on_sc=False),
    scratch_types=[
        pltpu.VMEM((3 * PER_W,), jnp.int32),
        pltpu.VMEM((NBUF, CHUNK, D), jnp.float32),
        pltpu.SemaphoreType.DMA((NBUF,)),
        pltpu.SemaphoreType.DMA((NBUF,)),
    ],
)(_gather_body)


def kernel(embs, idx0, idx1, idx2, p0, p1, p2, len0, len1, len2):
    o0, o1, o2 = _gather3(embs, idx0.reshape(N), idx1.reshape(N),
                          idx2.reshape(N))
    return (o1.reshape(L, B, D), p1, len1,
            o2.reshape(L, B, D), p2, len2,
            o0.reshape(L, B, D), p0, len0)
